# Initial kernel scaffold; baseline (speedup 1.0000x reference)
import jax
import jax.numpy as jnp
from jax.experimental import pallas as pl


def kernel(h_microbe, h_fmri, labels):
    raise NotImplementedError("write your pallas kernel here")


# speedup vs baseline: 42.6588x; 42.6588x over previous
import functools

import jax
import jax.numpy as jnp
from jax import lax
from jax.experimental import pallas as pl
from jax.experimental.pallas import tpu as pltpu

TEMP = 0.07
HARD_NEG_RATIO = 0.2
NUM_CLASSES = 10
R = 8
BISECT_ITERS = 32


CHUNK = 1024


def _loss_kernel(hm_ref, hft_ref, lab_col_ref, lab_row_ref, out_ref,
                 keys_ref, rowlse_ref):
    B = hm_ref.shape[0]
    nblk = B // R
    lab_col = lab_col_ref[...]

    sumsq = jnp.int32(0)
    for c in range(NUM_CLASSES):
        n_c = jnp.sum((lab_col == c).astype(jnp.int32))
        sumsq = sumsq + n_c * n_c
    neg_mean = (jnp.float32(B) * jnp.float32(B) - sumsq.astype(jnp.float32)) / jnp.float32(B)
    kk = jnp.floor(jnp.float32(HARD_NEG_RATIO) * neg_mean).astype(jnp.int32)

    def block_logits(b):
        hm_b = hm_ref[pl.ds(b * R, R), :]
        return lax.dot_general(hm_b, hft_ref[...], (((1,), (0,)), ((), ())),
                               preferred_element_type=jnp.float32) / TEMP

    def phase_a(b, carry):
        cmax, csum = carry
        logits = block_logits(b)
        rowmax = jnp.max(logits, axis=1, keepdims=True)
        rowsum = jnp.sum(jnp.exp(logits - rowmax), axis=1, keepdims=True)
        rowlse_ref[pl.ds(b * R, R), :] = rowmax + jnp.log(rowsum)

        bmax = jnp.max(logits, axis=0, keepdims=True)
        nmax = jnp.maximum(cmax, bmax)
        csum = csum * jnp.exp(cmax - nmax) + jnp.sum(jnp.exp(logits - nmax),
                                                     axis=0, keepdims=True)
        return nmax, csum

    cmax0 = jnp.full((1, B), -jnp.inf, dtype=jnp.float32)
    csum0 = jnp.zeros((1, B), dtype=jnp.float32)
    cmax, csum = lax.fori_loop(0, nblk, phase_a, (cmax0, csum0))
    collse = cmax + jnp.log(csum)

    chunk = keys_ref.shape[0]
    cblk = chunk // R

    def phase_b_chunk(ch, acc):
        blk0 = ch * cblk

        def write_keys(i, _):
            b = blk0 + i
            logits = block_logits(b)
            lab_b = lab_row_ref[pl.ds(b * R, R), :]
            negv = jnp.where(lab_b != lab_col, logits, 0.0)
            bits = lax.bitcast_convert_type(negv, jnp.int32)
            m = lax.shift_right_arithmetic(bits, 31) | jnp.int32(-2147483648)
            keys_ref[pl.ds(i * R, R), :] = lax.bitcast_convert_type(
                bits ^ m, jnp.uint32)
            return 0

        lax.fori_loop(0, cblk, write_keys, 0)

        def block_loss(i, acc):
            b = blk0 + i
            keys_b = keys_ref[pl.ds(i * R, R), :]

            def bisect(_, lh):
                lo, hi = lh
                mid = lo + lax.shift_right_logical(hi - lo, jnp.uint32(1))
                cnt = jnp.sum((keys_b > mid).astype(jnp.int32), axis=1,
                              keepdims=True)
                ge = cnt >= kk
                return jnp.where(ge, mid, lo), jnp.where(ge, hi, mid)

            lo0 = jnp.zeros((R, 1), dtype=jnp.uint32)
            hi0 = jnp.full((R, 1), jnp.uint32(0xFFFFFFFF))
            lo, _ = lax.fori_loop(0, BISECT_ITERS, bisect, (lo0, hi0))

            logits = block_logits(b)
            lab_b = lab_row_ref[pl.ds(b * R, R), :]
            pos = (lab_b == lab_col).astype(jnp.float32)
            w = jnp.where(keys_b > lo, 0.5, pos)
            rlse = rowlse_ref[pl.ds(b * R, R), :]
            terms = w * (2.0 * logits - rlse - collse)
            return acc + jnp.sum(terms)

        return lax.fori_loop(0, cblk, block_loss, acc)

    acc = lax.fori_loop(0, B // chunk, phase_b_chunk, jnp.float32(0.0))
    out_ref[...] = (-acc / (jnp.float32(B) * jnp.float32(B))).reshape(1, 1)


@jax.jit
def kernel(h_microbe, h_fmri, labels):
    B = h_microbe.shape[0]
    hft = h_fmri.T
    lab_col = labels.reshape(1, B).astype(jnp.int32)
    lab_row = labels.reshape(B, 1).astype(jnp.int32)
    out = pl.pallas_call(
        _loss_kernel,
        out_shape=jax.ShapeDtypeStruct((1, 1), jnp.float32),
        scratch_shapes=[
            pltpu.VMEM((min(CHUNK, B), B), jnp.uint32),
            pltpu.VMEM((B, 1), jnp.float32),
        ],
    )(h_microbe, hft, lab_col, lab_row)
    return out[0, 0]


# speedup vs baseline: 117.7186x; 2.7595x over previous
import jax
import jax.numpy as jnp
from jax import lax
from jax.experimental import pallas as pl
from jax.experimental.pallas import tpu as pltpu

TEMP = 0.07
HARD_NEG_RATIO = 0.2
NUM_CLASSES = 10
CB = 128
G = 256
CHUNK_L = 1024
BISECT_ITERS = 32
UNROLL = 8


def _loss_kernel(hf_ref, hmT_ref, oh_ref, ohT_ref, lab_ref, out_ref,
                 keysT_ref, rowlse_ref, colmax_ref, colsum_ref, lo_ref):
    B = hf_ref.shape[0]
    CHL = keysT_ref.shape[1]
    nlb = B // CB
    ngr = B // G
    lab = lab_ref[...]

    sumsq = jnp.int32(0)
    for c in range(NUM_CLASSES):
        n_c = jnp.sum((lab == c).astype(jnp.int32))
        sumsq = sumsq + n_c * n_c
    neg_mean = (jnp.float32(B) * jnp.float32(B) - sumsq.astype(jnp.float32)) / jnp.float32(B)
    kk = jnp.floor(jnp.float32(HARD_NEG_RATIO) * neg_mean).astype(jnp.int32)

    def lt_block(cb, g):
        return lax.dot_general(
            hf_ref[pl.ds(g * G, G), :], hmT_ref[:, pl.ds(cb * CB, CB)],
            (((1,), (0,)), ((), ())), preferred_element_type=jnp.float32) / TEMP

    def same_block(cb, g):
        return lax.dot_general(
            oh_ref[pl.ds(g * G, G), :], ohT_ref[:, pl.ds(cb * CB, CB)],
            (((1,), (0,)), ((), ())), preferred_element_type=jnp.float32)

    def p1_block(cb, _):
        def p1_group(g, carry):
            rmax, rsum = carry
            logits = lt_block(cb, g)
            gmax_r = jnp.max(logits, axis=0, keepdims=True)
            nmax = jnp.maximum(rmax, gmax_r)
            rsum = rsum * jnp.exp(rmax - nmax) + jnp.sum(
                jnp.exp(logits - nmax), axis=0, keepdims=True)

            gmax_c = jnp.max(logits, axis=1, keepdims=True)
            old = colmax_ref[pl.ds(g * G, G), :]
            colmax_ref[pl.ds(g * G, G), :] = jnp.where(
                cb == 0, gmax_c, jnp.maximum(old, gmax_c))
            return nmax, rsum

        rmax0 = jnp.full((1, CB), -jnp.inf, dtype=jnp.float32)
        rsum0 = jnp.zeros((1, CB), dtype=jnp.float32)
        rmax, rsum = lax.fori_loop(0, ngr, p1_group, (rmax0, rsum0))
        rowlse_ref[:, pl.ds(cb * CB, CB)] = rmax + jnp.log(rsum)
        return 0

    lax.fori_loop(0, nlb, p1_block, 0)

    ones_cb = jnp.ones((CB, 1), dtype=jnp.float32)

    def p2_block(cb, _):
        def p2_group(g, _g):
            logits = lt_block(cb, g)
            cmax = colmax_ref[pl.ds(g * G, G), :]
            e = jnp.exp(logits - cmax)
            part = lax.dot_general(e, ones_cb, (((1,), (0,)), ((), ())),
                                   preferred_element_type=jnp.float32)
            old = colsum_ref[pl.ds(g * G, G), :]
            colsum_ref[pl.ds(g * G, G), :] = jnp.where(cb == 0, part, old + part)
            return 0

        lax.fori_loop(0, ngr, p2_group, 0)
        return 0

    lax.fori_loop(0, nlb, p2_block, 0)

    clb = CHL // CB
    nrd = B // (8 * UNROLL)

    def p3_chunk(ch, acc):
        def write_keys(t, _):
            cb2 = t // ngr
            g = t % ngr
            logits = lt_block(ch * clb + cb2, g)
            negv = logits * (1.0 - same_block(ch * clb + cb2, g))
            bits = lax.bitcast_convert_type(negv, jnp.int32)
            m = lax.shift_right_arithmetic(bits, 31) | jnp.int32(-2147483648)
            keysT_ref[pl.ds(g * G, G), pl.ds(cb2 * CB, CB)] = (
                lax.bitcast_convert_type(bits ^ m, jnp.uint32))
            return 0

        lax.fori_loop(0, clb * ngr, write_keys, 0)

        def bis_cond(st):
            it, lo, hi, cntlo = st
            return jnp.logical_and(it < BISECT_ITERS,
                                   jnp.logical_not(jnp.all(cntlo == kk)))

        def bis_body(st):
            it, lo, hi, cntlo = st
            mid = lo + lax.shift_right_logical(hi - lo, jnp.uint32(1))

            def count_rows(r, acc8):
                base = r * 8 * UNROLL
                for u in range(UNROLL):
                    k8 = keysT_ref[pl.ds(base + u * 8, 8), :]
                    acc8 = acc8 + (k8 > mid).astype(jnp.int32)
                return acc8

            acc8 = lax.fori_loop(0, nrd, count_rows,
                                 jnp.zeros((8, CHL), dtype=jnp.int32))
            cnt = jnp.sum(acc8, axis=0, keepdims=True)
            ge = cnt >= kk
            return (it + 1, jnp.where(ge, mid, lo), jnp.where(ge, hi, mid),
                    jnp.where(ge, cnt, cntlo))

        lo0 = jnp.zeros((1, CHL), dtype=jnp.uint32)
        hi0 = jnp.full((1, CHL), jnp.uint32(0xFFFFFFFF))
        cnt0 = jnp.full((1, CHL), jnp.int32(-1))
        _, lo, _, _ = lax.while_loop(bis_cond, bis_body,
                                     (jnp.int32(0), lo0, hi0, cnt0))
        lo_ref[...] = lo

        def accum(t, acc):
            cb2 = t // ngr
            g = t % ngr
            cb = ch * clb + cb2
            logits = lt_block(cb, g)
            same = same_block(cb, g)
            keys_g = keysT_ref[pl.ds(g * G, G), pl.ds(cb2 * CB, CB)]
            w = jnp.where(keys_g > lo_ref[:, pl.ds(cb2 * CB, CB)], 0.5, same)
            rlse = rowlse_ref[:, pl.ds(cb * CB, CB)]
            clse = colmax_ref[pl.ds(g * G, G), :] + jnp.log(
                colsum_ref[pl.ds(g * G, G), :])
            terms = w * (2.0 * logits - rlse - clse)
            return acc + jnp.sum(terms)

        return lax.fori_loop(0, clb * ngr, accum, acc)

    acc = lax.fori_loop(0, B // CHL, p3_chunk, jnp.float32(0.0))
    out_ref[...] = (-acc / (jnp.float32(B) * jnp.float32(B))).reshape(1, 1)


@jax.jit
def kernel(h_microbe, h_fmri, labels):
    B = h_microbe.shape[0]
    oh = (labels[:, None] == jnp.arange(NUM_CLASSES)[None, :]).astype(jnp.float32)
    lab_col = labels.reshape(1, B).astype(jnp.int32)
    out = pl.pallas_call(
        _loss_kernel,
        out_shape=jax.ShapeDtypeStruct((1, 1), jnp.float32),
        scratch_shapes=[
            pltpu.VMEM((B, min(CHUNK_L, B)), jnp.uint32),
            pltpu.VMEM((1, B), jnp.float32),
            pltpu.VMEM((B, 1), jnp.float32),
            pltpu.VMEM((B, 1), jnp.float32),
            pltpu.VMEM((1, min(CHUNK_L, B)), jnp.uint32),
        ],
    )(h_fmri, h_microbe.T, oh, oh.T, lab_col)
    return out[0, 0]
